# baseline (speedup 1.0000x reference)
import jax
import jax.numpy as jnp
from jax.experimental import pallas as pl

H, W, NPOS = 512, 512, 64
BH = 256


def _body(img_ref, hp_ref, vp_ref, hb_ref, vb_ref, hpos_ref, vpos_ref,
          out_ref):
    i = pl.program_id(0)

    for c in range(3):
        out_ref[0, :, c, :] = img_ref[0, c]

    off = pl.multiple_of(i * BH, BH)
    hp = jnp.transpose(hp_ref[:, pl.ds(off, BH)])
    out_ref[0, :, 3, :] = jnp.broadcast_to(hp, (BH, W))
    vp = jnp.broadcast_to(vp_ref[...], (BH, W))
    out_ref[0, :, 4, :] = vp
    hb = jnp.transpose(hb_ref[:, pl.ds(off, BH)])
    out_ref[0, :, 5, :] = jnp.broadcast_to(hb, (BH, W))
    vb = jnp.broadcast_to(vb_ref[...], (BH, W))
    out_ref[0, :, 6, :] = vb

    row_ids = jax.lax.broadcasted_iota(jnp.int32, (NPOS, BH), 1)
    hmask = jnp.transpose(jnp.any(row_ids + i * BH == hpos_ref[...], axis=0,
                                  keepdims=True))
    col_ids = jax.lax.broadcasted_iota(jnp.int32, (NPOS, W), 1)
    vmask = jnp.any(col_ids == vpos_ref[...], axis=0,
                    keepdims=True)
    out_ref[0, :, 7, :] = jnp.maximum(hmask.astype(jnp.float32),
                                      vmask.astype(jnp.float32))


def kernel(normalized_image, h_probs, v_probs, h_binary, v_binary,
           h_positions, v_positions):
    imgp = jnp.transpose(normalized_image, (0, 3, 1, 2))
    hpos = h_positions.astype(jnp.int32)
    vpos = v_positions.astype(jnp.int32)

    out = pl.pallas_call(
        _body,
        grid=(H // BH,),
        in_specs=[
            pl.BlockSpec((1, 3, BH, W), lambda i: (0, 0, i, 0)),
            pl.BlockSpec((1, H), lambda i: (0, 0)),
            pl.BlockSpec((1, W), lambda i: (0, 0)),
            pl.BlockSpec((1, H), lambda i: (0, 0)),
            pl.BlockSpec((1, W), lambda i: (0, 0)),
            pl.BlockSpec((NPOS, 1), lambda i: (0, 0)),
            pl.BlockSpec((NPOS, 1), lambda i: (0, 0)),
        ],
        out_specs=pl.BlockSpec((1, BH, 8, W), lambda i: (0, i, 0, 0)),
        out_shape=jax.ShapeDtypeStruct((1, H, 8, W), jnp.float32),
    )(imgp, h_probs, v_probs, h_binary, v_binary, hpos, vpos)
    return jnp.transpose(out, (0, 1, 3, 2))


# speedup vs baseline: 1.4572x; 1.4572x over previous
import jax
import jax.numpy as jnp
from jax.experimental import pallas as pl

H, W, NPOS = 512, 512, 64
BH = 256


def _body(img_ref, hp_ref, vp_ref, hb_ref, vb_ref, hpos_ref, vpos_ref,
          out_ref):
    i = pl.program_id(0)

    for c in range(3):
        out_ref[0, :, c, :] = img_ref[0, c]

    off = pl.multiple_of(i * BH, BH)
    hp = jnp.transpose(hp_ref[:, pl.ds(off, BH)])
    out_ref[0, :, 3, :] = jnp.broadcast_to(hp, (BH, W))
    vp = jnp.broadcast_to(vp_ref[...], (BH, W))
    out_ref[0, :, 4, :] = vp
    hb = jnp.transpose(hb_ref[:, pl.ds(off, BH)])
    out_ref[0, :, 5, :] = jnp.broadcast_to(hb, (BH, W))
    vb = jnp.broadcast_to(vb_ref[...], (BH, W))
    out_ref[0, :, 6, :] = vb

    row_ids = jax.lax.broadcasted_iota(jnp.int32, (BH, NPOS), 0)
    hmask = jnp.any(row_ids + i * BH == hpos_ref[...], axis=1,
                    keepdims=True)
    col_ids = jax.lax.broadcasted_iota(jnp.int32, (NPOS, W), 1)
    vmask = jnp.any(col_ids == jnp.transpose(vpos_ref[...]), axis=0,
                    keepdims=True)
    out_ref[0, :, 7, :] = jnp.maximum(hmask.astype(jnp.float32),
                                      vmask.astype(jnp.float32))


def kernel(normalized_image, h_probs, v_probs, h_binary, v_binary,
           h_positions, v_positions):
    imgp = jnp.transpose(normalized_image, (0, 3, 1, 2))
    hpos = h_positions.astype(jnp.int32).reshape(1, NPOS)
    vpos = v_positions.astype(jnp.int32).reshape(1, NPOS)

    out = pl.pallas_call(
        _body,
        grid=(H // BH,),
        in_specs=[
            pl.BlockSpec((1, 3, BH, W), lambda i: (0, 0, i, 0)),
            pl.BlockSpec((1, H), lambda i: (0, 0)),
            pl.BlockSpec((1, W), lambda i: (0, 0)),
            pl.BlockSpec((1, H), lambda i: (0, 0)),
            pl.BlockSpec((1, W), lambda i: (0, 0)),
            pl.BlockSpec((1, NPOS), lambda i: (0, 0)),
            pl.BlockSpec((1, NPOS), lambda i: (0, 0)),
        ],
        out_specs=pl.BlockSpec((1, BH, 8, W), lambda i: (0, i, 0, 0)),
        out_shape=jax.ShapeDtypeStruct((1, H, 8, W), jnp.float32),
    )(imgp, h_probs, v_probs, h_binary, v_binary, hpos, vpos)
    return jnp.transpose(out, (0, 1, 3, 2))
